# baseline (speedup 1.0000x reference)
import functools

import jax
import jax.numpy as jnp
from jax import lax
from jax.experimental import pallas as pl
from jax.experimental.pallas import tpu as pltpu
from jax.experimental.pallas import tpu_sc as plsc

_SIZES = (256, 256, 128, 128, 128, 128, 64, 64, 64)
_OFFS = tuple(sum(_SIZES[:k]) for k in range(len(_SIZES)))
_B, _K, _T = 1024, 9, 50
_D = sum(_SIZES)
_NROWS = 1027
_NC, _NS = 2, 16
_NW = _NC * _NS
_BPW = _B // _NW
_CW = 64
_CHUNKS = tuple(
    (k, c * _CW, _OFFS[k] + c * _CW)
    for k in range(_K)
    for c in range(_SIZES[k] // _CW)
)
_TOFFS = (0, 16, 32, 34)

_mesh = plsc.VectorSubcoreMesh(
    core_axis_name="c", subcore_axis_name="s", num_cores=_NC, num_subcores=_NS
)


def _body(idx_hbm, e0, e1, e2, e3, e4, e5, e6, e7, e8, out,
          idxv, tchunk, tbuf, sem):
    tables = (e0, e1, e2, e3, e4, e5, e6, e7, e8)

    wid = lax.axis_index("s") * _NC + lax.axis_index("c")
    b0 = wid * _BPW
    pltpu.sync_copy(idx_hbm.at[pl.ds(b0, _BPW)], idxv)

    for k, coff, doff in _CHUNKS:
        pltpu.sync_copy(tables[k].at[:, pl.ds(coff, _CW)], tchunk)

        def b_body(bi, carry, k=k, doff=doff):
            ivecs = [idxv[bi, k, pl.ds(toff, 16)] for toff in _TOFFS]

            def s_body(s, c2):
                svec = jnp.full((16,), 0, jnp.int32) + s
                for g in range(4):
                    tbuf[s, pl.ds(_TOFFS[g], 16)] = plsc.load_gather(
                        tchunk, [ivecs[g], svec])
                return c2

            lax.fori_loop(0, _CW, s_body, 0)
            pltpu.sync_copy(tbuf, out.at[b0 + bi, pl.ds(doff, _CW), :])
            return carry

        lax.fori_loop(0, _BPW, b_body, 0)


@jax.jit
def kernel(indices, emb0, emb1, emb2, emb3, emb4, emb5, emb6, emb7, emb8):
    run = functools.partial(
        pl.kernel,
        out_type=jax.ShapeDtypeStruct((_B, _D, _T), jnp.float32),
        mesh=_mesh,
        scratch_types=[
            pltpu.VMEM((_BPW, _K, _T), jnp.int32),
            pltpu.VMEM((_NROWS, _CW), jnp.float32),
            pltpu.VMEM((_CW, _T), jnp.float32),
            pltpu.SemaphoreType.DMA,
        ],
        compiler_params=pltpu.CompilerParams(
            needs_layout_passes=False, use_tc_tiling_on_sc=False),
    )(_body)
    return run(indices, emb0, emb1, emb2, emb3, emb4, emb5, emb6, emb7, emb8)


# speedup vs baseline: 1.3891x; 1.3891x over previous
import functools

import jax
import jax.numpy as jnp
from jax import lax
from jax.experimental import pallas as pl
from jax.experimental.pallas import tpu as pltpu
from jax.experimental.pallas import tpu_sc as plsc

_SIZES = (256, 256, 128, 128, 128, 128, 64, 64, 64)
_OFFS = tuple(sum(_SIZES[:k]) for k in range(len(_SIZES)))
_B, _K, _T = 1024, 9, 50
_D = sum(_SIZES)
_NROWS = 1027
_NC, _NS = 2, 16
_NW = _NC * _NS
_BPW = _B // _NW
_CW = 64
_CP = 65
_CHUNKS = tuple(
    (k, c * _CW, _OFFS[k] + c * _CW)
    for k in range(_K)
    for c in range(_SIZES[k] // _CW)
)
_TOFFS = (0, 16, 32, 34)

_mesh = plsc.VectorSubcoreMesh(
    core_axis_name="c", subcore_axis_name="s", num_cores=_NC, num_subcores=_NS
)


def _body(idx_hbm, e0, e1, e2, e3, e4, e5, e6, e7, e8, out,
          idxv, tchunk, tbuf, sem):
    tables = (e0, e1, e2, e3, e4, e5, e6, e7, e8)

    wid = lax.axis_index("s") * _NC + lax.axis_index("c")
    b0 = wid * _BPW
    pltpu.sync_copy(idx_hbm.at[pl.ds(b0, _BPW)], idxv)

    for k, coff, doff in _CHUNKS:
        pltpu.sync_copy(tables[k].at[:, pl.ds(coff, _CW)],
                        tchunk.at[:, pl.ds(0, _CW)])

        def b_body(bi, carry, k=k, doff=doff):
            ivecs = [idxv[bi, k, pl.ds(toff, 16)] for toff in _TOFFS]

            def s_body(s, c2):
                svec = jnp.full((16,), 0, jnp.int32) + s
                for g in range(4):
                    tbuf[s, pl.ds(_TOFFS[g], 16)] = plsc.load_gather(
                        tchunk, [ivecs[g], svec])
                return c2

            lax.fori_loop(0, _CW, s_body, 0)
            pltpu.sync_copy(tbuf, out.at[b0 + bi, pl.ds(doff, _CW), :])
            return carry

        lax.fori_loop(0, _BPW, b_body, 0)


@jax.jit
def kernel(indices, emb0, emb1, emb2, emb3, emb4, emb5, emb6, emb7, emb8):
    run = functools.partial(
        pl.kernel,
        out_type=jax.ShapeDtypeStruct((_B, _D, _T), jnp.float32),
        mesh=_mesh,
        scratch_types=[
            pltpu.VMEM((_BPW, _K, _T), jnp.int32),
            pltpu.VMEM((_NROWS, _CP), jnp.float32),
            pltpu.VMEM((_CW, _T), jnp.float32),
            pltpu.SemaphoreType.DMA,
        ],
        compiler_params=pltpu.CompilerParams(
            needs_layout_passes=False, use_tc_tiling_on_sc=False),
    )(_body)
    return run(indices, emb0, emb1, emb2, emb3, emb4, emb5, emb6, emb7, emb8)


# speedup vs baseline: 1.4597x; 1.0508x over previous
import functools

import jax
import jax.numpy as jnp
from jax import lax
from jax.experimental import pallas as pl
from jax.experimental.pallas import tpu as pltpu
from jax.experimental.pallas import tpu_sc as plsc

_SIZES = (256, 256, 128, 128, 128, 128, 64, 64, 64)
_OFFS = tuple(sum(_SIZES[:k]) for k in range(len(_SIZES)))
_B, _K, _T = 1024, 9, 50
_D = sum(_SIZES)
_NROWS = 1027
_NC, _NS = 2, 16
_NW = _NC * _NS
_BPW = _B // _NW
_CW = 64
_CP = 65
_CHUNKS = tuple(
    (k, c * _CW, _OFFS[k] + c * _CW)
    for k in range(_K)
    for c in range(_SIZES[k] // _CW)
)
_TOFFS = (0, 16, 32, 34)

_mesh = plsc.VectorSubcoreMesh(
    core_axis_name="c", subcore_axis_name="s", num_cores=_NC, num_subcores=_NS
)


def _body(idx_hbm, e0, e1, e2, e3, e4, e5, e6, e7, e8, out,
          idxv, tchunk, tb0, tb1, sem0, sem1):
    tables = (e0, e1, e2, e3, e4, e5, e6, e7, e8)
    tbufs = (tb0, tb1)
    sems = (sem0, sem1)

    wid = lax.axis_index("s") * _NC + lax.axis_index("c")
    b0 = wid * _BPW
    pltpu.sync_copy(idx_hbm.at[pl.ds(b0, _BPW)], idxv)

    for k, coff, doff in _CHUNKS:
        pltpu.sync_copy(tables[k].at[:, pl.ds(coff, _CW)],
                        tchunk.at[:, pl.ds(0, _CW)])

        def fill(bi, p, k=k):
            tb = tbufs[p]
            ivecs = [idxv[bi, k, pl.ds(toff, 16)] for toff in _TOFFS]

            def s_body(s, c2):
                svec = jnp.full((16,), 0, jnp.int32) + s
                for g in range(4):
                    tb[s, pl.ds(_TOFFS[g], 16)] = plsc.load_gather(
                        tchunk, [ivecs[g], svec])
                return c2

            lax.fori_loop(0, _CW, s_body, 0)

        def start(bi, p, doff=doff):
            pltpu.async_copy(tbufs[p], out.at[b0 + bi, pl.ds(doff, _CW), :],
                             sems[p])

        def wait(p, doff=doff):
            pltpu.make_async_copy(out.at[b0, pl.ds(doff, _CW), :], tbufs[p],
                                  sems[p]).wait()

        fill(0, 0)
        start(0, 0)
        fill(1, 1)
        start(1, 1)

        def pair_body(j, carry):
            bi = 2 * j
            wait(0)
            fill(bi, 0)
            start(bi, 0)
            wait(1)
            fill(bi + 1, 1)
            start(bi + 1, 1)
            return carry

        lax.fori_loop(1, _BPW // 2, pair_body, 0)
        wait(0)
        wait(1)


@jax.jit
def kernel(indices, emb0, emb1, emb2, emb3, emb4, emb5, emb6, emb7, emb8):
    run = functools.partial(
        pl.kernel,
        out_type=jax.ShapeDtypeStruct((_B, _D, _T), jnp.float32),
        mesh=_mesh,
        scratch_types=[
            pltpu.VMEM((_BPW, _K, _T), jnp.int32),
            pltpu.VMEM((_NROWS, _CP), jnp.float32),
            pltpu.VMEM((_CW, _T), jnp.float32),
            pltpu.VMEM((_CW, _T), jnp.float32),
            pltpu.SemaphoreType.DMA,
            pltpu.SemaphoreType.DMA,
        ],
        compiler_params=pltpu.CompilerParams(
            needs_layout_passes=False, use_tc_tiling_on_sc=False),
    )(_body)
    return run(indices, emb0, emb1, emb2, emb3, emb4, emb5, emb6, emb7, emb8)


# speedup vs baseline: 1.4692x; 1.0065x over previous
import functools

import jax
import jax.numpy as jnp
from jax import lax
from jax.experimental import pallas as pl
from jax.experimental.pallas import tpu as pltpu
from jax.experimental.pallas import tpu_sc as plsc

_SIZES = (256, 256, 128, 128, 128, 128, 64, 64, 64)
_B, _K, _T = 1024, 9, 50
_D = sum(_SIZES)
_NROWS = 1027
_NC, _NS = 2, 16
_NW = _NC * _NS
_BPW = _B // _NW
_CW = 64
_CP = 65
_NCHUNK = _D // _CW
_KMAP = tuple(k for k in range(_K) for _ in range(_SIZES[k] // _CW))
_TOFFS = (0, 16, 32, 34)

_mesh = plsc.VectorSubcoreMesh(
    core_axis_name="c", subcore_axis_name="s", num_cores=_NC, num_subcores=_NS
)


def _body(cidx_hbm, cols_hbm, out, idxv, tchunk, tb0, tb1, sem0, sem1):
    tbufs = (tb0, tb1)
    sems = (sem0, sem1)

    wid = lax.axis_index("s") * _NC + lax.axis_index("c")
    b0 = wid * _BPW

    def chunk_body(j, carry):
        doff = j * _CW
        pltpu.sync_copy(cols_hbm.at[:, pl.ds(doff, _CW)],
                        tchunk.at[:, pl.ds(0, _CW)])
        pltpu.sync_copy(cidx_hbm.at[j, pl.ds(b0, _BPW)], idxv)

        def fill(bi, p):
            tb = tbufs[p]
            ivecs = [idxv[bi, pl.ds(toff, 16)] for toff in _TOFFS]

            def s_body(s, c2):
                svec = jnp.full((16,), 0, jnp.int32) + s
                for g in range(4):
                    tb[s, pl.ds(_TOFFS[g], 16)] = plsc.load_gather(
                        tchunk, [ivecs[g], svec])
                return c2

            lax.fori_loop(0, _CW, s_body, 0, unroll=8)

        def start(bi, p):
            pltpu.async_copy(tbufs[p], out.at[b0 + bi, pl.ds(doff, _CW), :],
                             sems[p])

        def wait(p):
            pltpu.make_async_copy(out.at[b0, pl.ds(doff, _CW), :], tbufs[p],
                                  sems[p]).wait()

        fill(0, 0)
        start(0, 0)
        fill(1, 1)
        start(1, 1)

        def pair_body(i, c2):
            bi = 2 * i
            wait(0)
            fill(bi, 0)
            start(bi, 0)
            wait(1)
            fill(bi + 1, 1)
            start(bi + 1, 1)
            return c2

        lax.fori_loop(1, _BPW // 2, pair_body, 0)
        wait(0)
        wait(1)
        return carry

    lax.fori_loop(0, _NCHUNK, chunk_body, 0)


@jax.jit
def kernel(indices, emb0, emb1, emb2, emb3, emb4, emb5, emb6, emb7, emb8):
    cols = jnp.concatenate(
        (emb0, emb1, emb2, emb3, emb4, emb5, emb6, emb7, emb8), axis=1)
    cidx = jnp.transpose(indices, (1, 0, 2))[jnp.array(_KMAP, jnp.int32)]
    run = functools.partial(
        pl.kernel,
        out_type=jax.ShapeDtypeStruct((_B, _D, _T), jnp.float32),
        mesh=_mesh,
        scratch_types=[
            pltpu.VMEM((_BPW, _T), jnp.int32),
            pltpu.VMEM((_NROWS, _CP), jnp.float32),
            pltpu.VMEM((_CW, _T), jnp.float32),
            pltpu.VMEM((_CW, _T), jnp.float32),
            pltpu.SemaphoreType.DMA,
            pltpu.SemaphoreType.DMA,
        ],
        compiler_params=pltpu.CompilerParams(
            needs_layout_passes=False, use_tc_tiling_on_sc=False),
    )(_body)
    return run(cidx, cols)


# speedup vs baseline: 1.4743x; 1.0034x over previous
import functools

import jax
import jax.numpy as jnp
from jax import lax
from jax.experimental import pallas as pl
from jax.experimental.pallas import tpu as pltpu
from jax.experimental.pallas import tpu_sc as plsc

_SIZES = (256, 256, 128, 128, 128, 128, 64, 64, 64)
_B, _K, _T = 1024, 9, 50
_D = sum(_SIZES)
_NROWS = 1027
_NC, _NS = 2, 16
_NW = _NC * _NS
_BPW = _B // _NW
_CW = 64
_CP = 65
_DOFF = tuple(sum(_SIZES[:k]) for k in range(_K))
_TOFFS = (0, 16, 32, 34)

_mesh = plsc.VectorSubcoreMesh(
    core_axis_name="c", subcore_axis_name="s", num_cores=_NC, num_subcores=_NS
)


def _body(idx_hbm, *rest):
    embs = rest[:_K]
    out = rest[_K]
    idxv, tchunk, tb0, tb1, sem0, sem1 = rest[_K + 1:]
    tbufs = (tb0, tb1)
    sems = (sem0, sem1)

    wid = lax.axis_index("s") * _NC + lax.axis_index("c")
    b0 = wid * _BPW

    pltpu.sync_copy(idx_hbm.at[pl.ds(b0, _BPW)], idxv)

    for k in range(_K):
        emb = embs[k]
        base = _DOFF[k]

        def chunk_body(c, carry):
            coff = c * _CW
            doff = base + coff
            pltpu.sync_copy(emb.at[:, pl.ds(coff, _CW)],
                            tchunk.at[:, pl.ds(0, _CW)])

            def fill(bi, p):
                tb = tbufs[p]
                ivecs = [idxv[bi, k, pl.ds(toff, 16)] for toff in _TOFFS]

                def s_body(s, c2):
                    svec = jnp.full((16,), 0, jnp.int32) + s
                    for g in range(4):
                        tb[s, pl.ds(_TOFFS[g], 16)] = plsc.load_gather(
                            tchunk, [ivecs[g], svec])
                    return c2

                lax.fori_loop(0, _CW, s_body, 0, unroll=4)

            def start(bi, p):
                pltpu.async_copy(tbufs[p],
                                 out.at[b0 + bi, pl.ds(doff, _CW), :],
                                 sems[p])

            def wait(p):
                pltpu.make_async_copy(out.at[b0, pl.ds(doff, _CW), :],
                                      tbufs[p], sems[p]).wait()

            pltpu.async_copy(out.at[b0, pl.ds(doff, _CW), :], tb0, sem0)
            pltpu.async_copy(out.at[b0, pl.ds(doff, _CW), :], tb1, sem1)

            def pair_body(i, c2):
                bi = 2 * i
                wait(0)
                fill(bi, 0)
                start(bi, 0)
                wait(1)
                fill(bi + 1, 1)
                start(bi + 1, 1)
                return c2

            lax.fori_loop(0, _BPW // 2, pair_body, 0)
            wait(0)
            wait(1)
            return carry

        lax.fori_loop(0, _SIZES[k] // _CW, chunk_body, 0)


@jax.jit
def kernel(indices, emb0, emb1, emb2, emb3, emb4, emb5, emb6, emb7, emb8):
    run = functools.partial(
        pl.kernel,
        out_type=jax.ShapeDtypeStruct((_B, _D, _T), jnp.float32),
        mesh=_mesh,
        scratch_types=[
            pltpu.VMEM((_BPW, _K, _T), jnp.int32),
            pltpu.VMEM((_NROWS, _CP), jnp.float32),
            pltpu.VMEM((_CW, _T), jnp.float32),
            pltpu.VMEM((_CW, _T), jnp.float32),
            pltpu.SemaphoreType.DMA,
            pltpu.SemaphoreType.DMA,
        ],
        compiler_params=pltpu.CompilerParams(
            needs_layout_passes=False, use_tc_tiling_on_sc=False),
    )(_body)
    return run(indices, emb0, emb1, emb2, emb3, emb4, emb5, emb6, emb7, emb8)


# speedup vs baseline: 2.0784x; 1.4097x over previous
import functools

import jax
import jax.numpy as jnp
from jax import lax
from jax.experimental import pallas as pl
from jax.experimental.pallas import tpu as pltpu
from jax.experimental.pallas import tpu_sc as plsc

_SIZES = (256, 256, 128, 128, 128, 128, 64, 64, 64)
_B, _K, _T = 1024, 9, 50
_D = sum(_SIZES)
_NROWS = 1027
_NC, _NS = 2, 16
_NW = _NC * _NS
_BPW = _B // _NW
_CW = 64
_CP = 65
_TPAD = 128
_TS = 64
_DOFF = tuple(sum(_SIZES[:k]) for k in range(_K))
_TOFFS = (0, 16, 32, 34)

_mesh = plsc.VectorSubcoreMesh(
    core_axis_name="c", subcore_axis_name="s", num_cores=_NC, num_subcores=_NS
)


def _body(idx_hbm, *rest):
    embs = rest[:_K]
    out = rest[_K]
    idxv, tchunk, tb0, tb1, sem0, sem1 = rest[_K + 1:]
    tbufs = (tb0, tb1)
    sems = (sem0, sem1)

    wid = lax.axis_index("s") * _NC + lax.axis_index("c")
    b0 = wid * _BPW

    pltpu.sync_copy(idx_hbm.at[pl.ds(b0, _BPW)], idxv)

    for k in range(_K):
        emb = embs[k]
        base = _DOFF[k]

        def chunk_body(c, carry):
            coff = c * _CW
            doff = base + coff
            pltpu.sync_copy(emb.at[:, pl.ds(coff, _CW)],
                            tchunk.at[:, pl.ds(0, _CW)])

            def fill(bi, p):
                tb = tbufs[p]
                ivecs = [idxv[bi, k, pl.ds(toff, 16)] for toff in _TOFFS]

                def s_body(s, c2):
                    svec = jnp.full((16,), 0, jnp.int32) + s
                    for g in range(4):
                        tb[s, pl.ds(_TOFFS[g], 16)] = plsc.load_gather(
                            tchunk, [ivecs[g], svec])
                    return c2

                lax.fori_loop(0, _CW, s_body, 0, unroll=4)

            def start(bi, p):
                pltpu.async_copy(tbufs[p],
                                 out.at[b0 + bi, pl.ds(doff, _CW),
                                        pl.ds(0, _TS)],
                                 sems[p])

            def wait(p):
                pltpu.make_async_copy(out.at[b0, pl.ds(doff, _CW),
                                             pl.ds(0, _TS)],
                                      tbufs[p], sems[p]).wait()

            pltpu.async_copy(out.at[b0, pl.ds(doff, _CW), pl.ds(0, _TS)],
                             tb0, sem0)
            pltpu.async_copy(out.at[b0, pl.ds(doff, _CW), pl.ds(0, _TS)],
                             tb1, sem1)

            def pair_body(i, c2):
                bi = 2 * i
                wait(0)
                fill(bi, 0)
                start(bi, 0)
                wait(1)
                fill(bi + 1, 1)
                start(bi + 1, 1)
                return c2

            lax.fori_loop(0, _BPW // 2, pair_body, 0)
            wait(0)
            wait(1)
            return carry

        lax.fori_loop(0, _SIZES[k] // _CW, chunk_body, 0)


@jax.jit
def kernel(indices, emb0, emb1, emb2, emb3, emb4, emb5, emb6, emb7, emb8):
    run = functools.partial(
        pl.kernel,
        out_type=jax.ShapeDtypeStruct((_B, _D, _TPAD), jnp.float32),
        mesh=_mesh,
        scratch_types=[
            pltpu.VMEM((_BPW, _K, _T), jnp.int32),
            pltpu.VMEM((_NROWS, _CP), jnp.float32),
            pltpu.VMEM((_CW, _TS), jnp.float32),
            pltpu.VMEM((_CW, _TS), jnp.float32),
            pltpu.SemaphoreType.DMA,
            pltpu.SemaphoreType.DMA,
        ],
        compiler_params=pltpu.CompilerParams(
            needs_layout_passes=False, use_tc_tiling_on_sc=False),
    )(_body)
    out = run(indices, emb0, emb1, emb2, emb3, emb4, emb5, emb6, emb7, emb8)
    return out[:, :, :_T]


# speedup vs baseline: 2.0912x; 1.0062x over previous
import functools

import jax
import jax.numpy as jnp
from jax import lax
from jax.experimental import pallas as pl
from jax.experimental.pallas import tpu as pltpu
from jax.experimental.pallas import tpu_sc as plsc

_SIZES = (256, 256, 128, 128, 128, 128, 64, 64, 64)
_B, _K, _T = 1024, 9, 50
_D = sum(_SIZES)
_NROWS = 1027
_NC, _NS = 2, 16
_NW = _NC * _NS
_BPW = _B // _NW
_CW = 64
_CP = 65
_TPAD = 128
_TS = 56
_DOFF = tuple(sum(_SIZES[:k]) for k in range(_K))
_TOFFS = (0, 16, 32, 34)

_mesh = plsc.VectorSubcoreMesh(
    core_axis_name="c", subcore_axis_name="s", num_cores=_NC, num_subcores=_NS
)


def _body(idx_hbm, *rest):
    embs = rest[:_K]
    out = rest[_K]
    idxv, tchunk, tb0, tb1, sem0, sem1 = rest[_K + 1:]
    tbufs = (tb0, tb1)
    sems = (sem0, sem1)

    wid = lax.axis_index("s") * _NC + lax.axis_index("c")
    b0 = wid * _BPW

    pltpu.sync_copy(idx_hbm.at[pl.ds(b0, _BPW)], idxv)

    for k in range(_K):
        emb = embs[k]
        base = _DOFF[k]

        def chunk_body(c, carry):
            coff = c * _CW
            doff = base + coff
            pltpu.sync_copy(emb.at[:, pl.ds(coff, _CW)],
                            tchunk.at[:, pl.ds(0, _CW)])

            def fill(bi, p):
                tb = tbufs[p]
                ivecs = [idxv[bi, k, pl.ds(toff, 16)] for toff in _TOFFS]

                def s_body(s, c2):
                    svec = jnp.full((16,), 0, jnp.int32) + s
                    for g in range(4):
                        tb[s, pl.ds(_TOFFS[g], 16)] = plsc.load_gather(
                            tchunk, [ivecs[g], svec])
                    return c2

                lax.fori_loop(0, _CW, s_body, 0, unroll=8)

            def start(bi, p):
                pltpu.async_copy(tbufs[p],
                                 out.at[b0 + bi, pl.ds(doff, _CW),
                                        pl.ds(0, _TS)],
                                 sems[p])

            def wait(p):
                pltpu.make_async_copy(out.at[b0, pl.ds(doff, _CW),
                                             pl.ds(0, _TS)],
                                      tbufs[p], sems[p]).wait()

            pltpu.async_copy(out.at[b0, pl.ds(doff, _CW), pl.ds(0, _TS)],
                             tb0, sem0)
            pltpu.async_copy(out.at[b0, pl.ds(doff, _CW), pl.ds(0, _TS)],
                             tb1, sem1)

            def pair_body(i, c2):
                bi = 2 * i
                wait(0)
                fill(bi, 0)
                start(bi, 0)
                wait(1)
                fill(bi + 1, 1)
                start(bi + 1, 1)
                return c2

            lax.fori_loop(0, _BPW // 2, pair_body, 0)
            wait(0)
            wait(1)
            return carry

        lax.fori_loop(0, _SIZES[k] // _CW, chunk_body, 0)


@jax.jit
def kernel(indices, emb0, emb1, emb2, emb3, emb4, emb5, emb6, emb7, emb8):
    run = functools.partial(
        pl.kernel,
        out_type=jax.ShapeDtypeStruct((_B, _D, _TPAD), jnp.float32),
        mesh=_mesh,
        scratch_types=[
            pltpu.VMEM((_BPW, _K, _T), jnp.int32),
            pltpu.VMEM((_NROWS, _CP), jnp.float32),
            pltpu.VMEM((_CW, _TS), jnp.float32),
            pltpu.VMEM((_CW, _TS), jnp.float32),
            pltpu.SemaphoreType.DMA,
            pltpu.SemaphoreType.DMA,
        ],
        compiler_params=pltpu.CompilerParams(
            needs_layout_passes=False, use_tc_tiling_on_sc=False),
    )(_body)
    out = run(indices, emb0, emb1, emb2, emb3, emb4, emb5, emb6, emb7, emb8)
    return out[:, :, :_T]
